# baseline (speedup 1.0000x reference)
import functools

import jax
import jax.numpy as jnp
from jax import lax
from jax.experimental import pallas as pl
from jax.experimental.pallas import tpu as pltpu
from jax.experimental.pallas import tpu_sc as plsc

N_NODES = 10000
N_EDGES = 320000
HIDDEN = 64
UW = 128
EDGE_DIM = 2
EP = 8
LATENT = 32

NW = 32
NE_PAD = 327680
NPAD = 10240
PER_TILE = NE_PAD // NW
CH = 512
NCH = CH // 128
N_OUTER = PER_TILE // CH
CNT_W = 16

_EPS = 1e-5
_PREC = lax.Precision.HIGHEST


def _relu(v):
    return jnp.maximum(v, 0.0)


def _dot(a, b):
    return jnp.dot(a, b, precision=_PREC, preferred_element_type=jnp.float32)


def _ln_rows(t, g, b, width):
    s = jnp.sum(t, axis=-1, keepdims=True)
    mean = s / width
    var = jnp.sum(t * t, axis=-1, keepdims=True) / width - mean * mean
    return (t - mean) * lax.rsqrt(var + _EPS) * g + b


def _node_enc_body(x_ref, w1_ref, b1_ref, w2_ref, b2_ref, g_ref, bb_ref,
                   m1b1_ref, m1w2_ref, m1b2_ref, eg_ref, ebb_ref,
                   u_ref, e0_ref):
    x = x_ref[...]
    h = _relu(_dot(x, w1_ref[...]) + b1_ref[...])
    t = _dot(h, w2_ref[...]) + b2_ref[...]
    u_ref[...] = _ln_rows(t, g_ref[...], bb_ref[...], HIDDEN)
    h0 = _relu(m1b1_ref[...])
    e0 = _dot(h0, m1w2_ref[...]) + m1b2_ref[...]
    e0_ref[...] = _ln_rows(e0, eg_ref[...], ebb_ref[...], EDGE_DIM)


def _edge_body_common(gs, gd, e,
                      wa_ref, wb_ref, wc_ref, b1_ref, w2_ref, b2_ref,
                      ea_ref, eb_ref, ec_ref, f1_ref, ew2_ref, f2_ref,
                      g_ref, bb_ref, m_ref, eo_ref):
    pre = _dot(gs, wa_ref[...]) + _dot(gd, wb_ref[...]) + _dot(e, wc_ref[...]) + b1_ref[...]
    m_ref[...] = _dot(_relu(pre), w2_ref[...]) + b2_ref[...]
    pre2 = _dot(gs, ea_ref[...]) + _dot(gd, eb_ref[...]) + _dot(e, ec_ref[...]) + f1_ref[...]
    q = _dot(_relu(pre2), ew2_ref[...]) + f2_ref[...]
    eo_ref[...] = _ln_rows(e + q, g_ref[...], bb_ref[...], EDGE_DIM)


def _edge_body(gs_ref, gd_ref, e_ref, *rest):
    _edge_body_common(gs_ref[...], gd_ref[...], e_ref[...], *rest)


def _edge0_body(gs_ref, gd_ref, e0_ref, *rest):
    e = jnp.broadcast_to(e0_ref[...], (gs_ref.shape[0], EP))
    _edge_body_common(gs_ref[...], gd_ref[...], e, *rest)


def _update_body(u_ref, p0_ref, p1_ref, c0_ref, c1_ref,
                 ua_ref, ub_ref, b1_ref, w2_ref, b2_ref, g_ref, bb_ref,
                 out_ref):
    u = u_ref[...]
    agg = p0_ref[...] + p1_ref[...]
    cnt = jnp.maximum(c0_ref[...][:, 0:1] + c1_ref[...][:, 0:1], 1.0)
    mean_agg = agg / cnt
    h = _relu(_dot(u, ua_ref[...]) + _dot(mean_agg, ub_ref[...]) + b1_ref[...])
    t = u + _dot(h, w2_ref[...]) + b2_ref[...]
    out_ref[...] = _ln_rows(t, g_ref[...], bb_ref[...], HIDDEN)


def _final_body(u_ref, w1_ref, b1_ref, w2_ref, b2_ref, g_ref, bb_ref,
                ow1_ref, ob1_ref, ow2_ref, ob2_ref, z_ref):
    u = u_ref[...]
    h = _relu(_dot(u, w1_ref[...]) + b1_ref[...])
    u3 = _ln_rows(_dot(h, w2_ref[...]) + b2_ref[...], g_ref[...], bb_ref[...], HIDDEN)
    z = jnp.mean(u3, axis=0, keepdims=True)
    h2 = _relu(_dot(z, ow1_ref[...]) + ob1_ref[...])
    z_ref[...] = _dot(h2, ow2_ref[...]) + ob2_ref[...]


def _full_spec(shape):
    return pl.BlockSpec(shape, lambda *_: tuple(0 for _ in shape))


def _node_enc(x, w):
    specs = [_full_spec(a.shape) for a in (x,) + w]
    return pl.pallas_call(
        _node_enc_body,
        grid=(1,),
        in_specs=specs,
        out_specs=[_full_spec((N_NODES, UW)), _full_spec((1, EP))],
        out_shape=[jax.ShapeDtypeStruct((N_NODES, UW), jnp.float32),
                   jax.ShapeDtypeStruct((1, EP), jnp.float32)],
    )(x, *w)


_BE = 2048


def _edge_stage(gs, gd, e, w, first):
    body = _edge0_body if first else _edge_body
    nblk = NE_PAD // _BE
    e_spec = (_full_spec((1, EP)) if first
              else pl.BlockSpec((_BE, EP), lambda i: (i, 0)))
    in_specs = [pl.BlockSpec((_BE, UW), lambda i: (i, 0)),
                pl.BlockSpec((_BE, UW), lambda i: (i, 0)),
                e_spec] + [_full_spec(a.shape) for a in w]
    return pl.pallas_call(
        body,
        grid=(nblk,),
        in_specs=in_specs,
        out_specs=[pl.BlockSpec((_BE, HIDDEN), lambda i: (i, 0)),
                   pl.BlockSpec((_BE, EP), lambda i: (i, 0))],
        out_shape=[jax.ShapeDtypeStruct((NE_PAD, HIDDEN), jnp.float32),
                   jax.ShapeDtypeStruct((NE_PAD, EP), jnp.float32)],
    )(gs, gd, e, *w)


_BN = 1000


def _update_stage(u, p, c0, c1, w):
    nblk = N_NODES // _BN
    in_specs = [pl.BlockSpec((_BN, UW), lambda i: (i, 0)),
                pl.BlockSpec((_BN, HIDDEN), lambda i: (i, 0)),
                pl.BlockSpec((_BN, HIDDEN), lambda i: (i, 0)),
                pl.BlockSpec((_BN, HIDDEN), lambda i: (i, 0)),
                pl.BlockSpec((_BN, HIDDEN), lambda i: (i, 0))] + \
               [_full_spec(a.shape) for a in w]
    return pl.pallas_call(
        _update_body,
        grid=(nblk,),
        in_specs=in_specs,
        out_specs=pl.BlockSpec((_BN, UW), lambda i: (i, 0)),
        out_shape=jax.ShapeDtypeStruct((N_NODES, UW), jnp.float32),
    )(u, p[0], p[1], c0, c1, *w)


def _final_stage(u, w):
    specs = [_full_spec(a.shape) for a in (u,) + w]
    return pl.pallas_call(
        _final_body,
        grid=(1,),
        in_specs=specs,
        out_specs=_full_spec((1, LATENT)),
        out_shape=jax.ShapeDtypeStruct((1, LATENT), jnp.float32),
    )(u, *w)


def _sc_mesh():
    return plsc.VectorSubcoreMesh(core_axis_name="c", subcore_axis_name="s",
                                  num_cores=2, num_subcores=16)


@functools.cache
def _sc_gather_kernel():
    @functools.partial(
        pl.kernel,
        out_type=(jax.ShapeDtypeStruct((NE_PAD, UW), jnp.float32),
                  jax.ShapeDtypeStruct((NE_PAD, UW), jnp.float32)),
        mesh=_sc_mesh(),
        scratch_types=[pltpu.VMEM((CH, UW), jnp.float32),
                       pltpu.VMEM((NCH, 128), jnp.int32),
                       pltpu.SemaphoreType.DMA],
    )
    def k(u_hbm, src_hbm, dst_hbm, gs_hbm, gd_hbm, vbuf, ibuf, sem):
        c = lax.axis_index("c")
        s = lax.axis_index("s")
        base_e = (c * 16 + s) * PER_TILE

        @pl.loop(0, N_OUTER)
        def _(j):
            b = pl.multiple_of(base_e + j * CH, CH)
            for idx_hbm, out_hbm in ((src_hbm, gs_hbm), (dst_hbm, gd_hbm)):
                pltpu.sync_copy(idx_hbm.at[pl.ds(pl.multiple_of(b // 128, NCH), NCH)], ibuf)
                copies = [
                    pltpu.async_copy(u_hbm.at[ibuf.at[jj]],
                                     vbuf.at[pl.ds(jj * 128, 128)], sem)
                    for jj in range(NCH)
                ]
                for cp in copies:
                    cp.wait()
                pltpu.sync_copy(vbuf, out_hbm.at[pl.ds(b, CH)])

    return k


def _sc_gather(u, src2, dst2):
    return _sc_gather_kernel()(u, src2, dst2)


def _seg_sum(vals, idx):
    half = NE_PAD // 2
    p0 = jax.ops.segment_sum(vals[:half], idx[:half], num_segments=NPAD)
    p1 = jax.ops.segment_sum(vals[half:], idx[half:], num_segments=NPAD)
    return p0, p1


def _row(v, width=None):
    v = v.reshape(1, -1)
    if width is not None and v.shape[1] < width:
        v = jnp.pad(v, ((0, 0), (0, width - v.shape[1])))
    return v


def _pad_rows(w, rows):
    return jnp.pad(w, ((0, rows - w.shape[0]), (0, 0)))


def _pad_cols(w, cols):
    return jnp.pad(w, ((0, 0), (0, cols - w.shape[1])))


def kernel(x, edge_index, params):
    p = params
    npad = NE_PAD - N_EDGES
    src = jnp.concatenate([edge_index[0], jnp.zeros((npad,), jnp.int32)])
    dst = jnp.concatenate([edge_index[1],
                           jnp.full((npad,), N_NODES, jnp.int32)])
    src2 = src.reshape(NE_PAD // 128, 128)
    dst2 = dst.reshape(NE_PAD // 128, 128)

    ones_h = jnp.ones((NE_PAD, HIDDEN), jnp.float32)

    enc_w = (p['mlp0_w1'], _row(p['mlp0_b1']),
             _pad_cols(p['mlp0_w2'], UW), _row(p['mlp0_b2'], UW),
             _row(p['lnm0_g'], UW), _row(p['lnm0_b'], UW),
             _row(p['mlp1_b1']), _pad_cols(p['mlp1_w2'], EP),
             _row(p['mlp1_b2'], EP), _row(p['lnm1_g'], EP), _row(p['lnm1_b'], EP))

    u, e0 = _node_enc(x, enc_w)

    _idx = dst2.reshape(-1)
    c0, c1 = _seg_sum(ones_h, _idx)

    e = e0
    for li, lyr in enumerate(p['layers']):
        gs, gd = _sc_gather(u, src2, dst2)
        ew = (_pad_rows(lyr['msg_w1'][:HIDDEN], UW),
              _pad_rows(lyr['msg_w1'][HIDDEN:2 * HIDDEN], UW),
              _pad_rows(lyr['msg_w1'][2 * HIDDEN:], EP),
              _row(lyr['msg_b1']), lyr['msg_w2'], _row(lyr['msg_b2']),
              _pad_rows(lyr['edg_w1'][:HIDDEN], UW),
              _pad_rows(lyr['edg_w1'][HIDDEN:2 * HIDDEN], UW),
              _pad_rows(lyr['edg_w1'][2 * HIDDEN:], EP),
              _row(lyr['edg_b1']), _pad_cols(lyr['edg_w2'], EP),
              _row(lyr['edg_b2'], EP),
              _row(lyr['ln_e_g'], EP), _row(lyr['ln_e_b'], EP))
        m, e = _edge_stage(gs, gd, e, ew, first=(li == 0))
        p0, p1 = _seg_sum(m, _idx)
        uw = (_pad_rows(lyr['upd_w1'][:HIDDEN], UW), lyr['upd_w1'][HIDDEN:],
              _row(lyr['upd_b1']), _pad_cols(lyr['upd_w2'], UW),
              _row(lyr['upd_b2'], UW),
              _row(lyr['ln_n_g'], UW), _row(lyr['ln_n_b'], UW))
        u = _update_stage(u, (p0, p1), c0, c1, uw)

    fw = (_pad_rows(p['mlp2_w1'], UW), _row(p['mlp2_b1']),
          p['mlp2_w2'], _row(p['mlp2_b2']),
          _row(p['lnm2_g']), _row(p['lnm2_b']),
          p['out_w1'], _row(p['out_b1']), p['out_w2'], _row(p['out_b2']))
    return _final_stage(u, fw)


# speedup vs baseline: 1.0253x; 1.0253x over previous
import functools

import jax
import jax.numpy as jnp
from jax import lax
from jax.experimental import pallas as pl
from jax.experimental.pallas import tpu as pltpu
from jax.experimental.pallas import tpu_sc as plsc

N_NODES = 10000
N_EDGES = 320000
HIDDEN = 64
UW = 128
EDGE_DIM = 2
EP = 8
LATENT = 32

NW = 32
NE_PAD = 327680
NPAD = 10240
PER_TILE = NE_PAD // NW
CH = 512
NCH = CH // 128
N_OUTER = PER_TILE // CH
CNT_W = 16

_EPS = 1e-5
_PREC = lax.Precision.HIGHEST


def _relu(v):
    return jnp.maximum(v, 0.0)


def _dot(a, b):
    return jnp.dot(a, b, precision=_PREC, preferred_element_type=jnp.float32)


def _ln_rows(t, g, b, width):
    s = jnp.sum(t, axis=-1, keepdims=True)
    mean = s / width
    var = jnp.sum(t * t, axis=-1, keepdims=True) / width - mean * mean
    return (t - mean) * lax.rsqrt(var + _EPS) * g + b


def _node_enc_body(x_ref, w1_ref, b1_ref, w2_ref, b2_ref, g_ref, bb_ref,
                   m1b1_ref, m1w2_ref, m1b2_ref, eg_ref, ebb_ref,
                   u_ref, e0_ref):
    x = x_ref[...]
    h = _relu(_dot(x, w1_ref[...]) + b1_ref[...])
    t = _dot(h, w2_ref[...]) + b2_ref[...]
    u_ref[...] = _ln_rows(t, g_ref[...], bb_ref[...], HIDDEN)
    h0 = _relu(m1b1_ref[...])
    e0 = _dot(h0, m1w2_ref[...]) + m1b2_ref[...]
    e0_ref[...] = _ln_rows(e0, eg_ref[...], ebb_ref[...], EDGE_DIM)


def _edge_body_common(gs, gd, e,
                      wa_ref, wb_ref, wc_ref, b1_ref, w2_ref, b2_ref,
                      ea_ref, eb_ref, ec_ref, f1_ref, ew2_ref, f2_ref,
                      g_ref, bb_ref, m_ref, eo_ref):
    pre = _dot(gs, wa_ref[...]) + _dot(gd, wb_ref[...]) + _dot(e, wc_ref[...]) + b1_ref[...]
    m_ref[...] = _dot(_relu(pre), w2_ref[...]) + b2_ref[...]
    pre2 = _dot(gs, ea_ref[...]) + _dot(gd, eb_ref[...]) + _dot(e, ec_ref[...]) + f1_ref[...]
    q = _dot(_relu(pre2), ew2_ref[...]) + f2_ref[...]
    eo_ref[...] = _ln_rows(e + q, g_ref[...], bb_ref[...], EDGE_DIM)


def _edge_body(gs_ref, gd_ref, e_ref, *rest):
    _edge_body_common(gs_ref[...], gd_ref[...], e_ref[...], *rest)


def _edge0_body(gs_ref, gd_ref, e0_ref, *rest):
    e = jnp.broadcast_to(e0_ref[...], (gs_ref.shape[0], EP))
    _edge_body_common(gs_ref[...], gd_ref[...], e, *rest)


def _update_body(u_ref, p0_ref, p1_ref, c0_ref, c1_ref,
                 ua_ref, ub_ref, b1_ref, w2_ref, b2_ref, g_ref, bb_ref,
                 out_ref):
    u = u_ref[...]
    agg = p0_ref[...] + p1_ref[...]
    cnt = jnp.maximum(c0_ref[...][:, 0:1] + c1_ref[...][:, 0:1], 1.0)
    mean_agg = agg / cnt
    h = _relu(_dot(u, ua_ref[...]) + _dot(mean_agg, ub_ref[...]) + b1_ref[...])
    t = u + _dot(h, w2_ref[...]) + b2_ref[...]
    out_ref[...] = _ln_rows(t, g_ref[...], bb_ref[...], HIDDEN)


def _final_body(u_ref, w1_ref, b1_ref, w2_ref, b2_ref, g_ref, bb_ref,
                ow1_ref, ob1_ref, ow2_ref, ob2_ref, z_ref):
    u = u_ref[...]
    h = _relu(_dot(u, w1_ref[...]) + b1_ref[...])
    u3 = _ln_rows(_dot(h, w2_ref[...]) + b2_ref[...], g_ref[...], bb_ref[...], HIDDEN)
    z = jnp.mean(u3, axis=0, keepdims=True)
    h2 = _relu(_dot(z, ow1_ref[...]) + ob1_ref[...])
    z_ref[...] = _dot(h2, ow2_ref[...]) + ob2_ref[...]


def _full_spec(shape):
    return pl.BlockSpec(shape, lambda *_: tuple(0 for _ in shape))


def _node_enc(x, w):
    specs = [_full_spec(a.shape) for a in (x,) + w]
    return pl.pallas_call(
        _node_enc_body,
        grid=(1,),
        in_specs=specs,
        out_specs=[_full_spec((N_NODES, UW)), _full_spec((1, EP))],
        out_shape=[jax.ShapeDtypeStruct((N_NODES, UW), jnp.float32),
                   jax.ShapeDtypeStruct((1, EP), jnp.float32)],
    )(x, *w)


_BE = 2048


def _edge_stage(gs, gd, e, w, first):
    body = _edge0_body if first else _edge_body
    nblk = NE_PAD // _BE
    e_spec = (_full_spec((1, EP)) if first
              else pl.BlockSpec((_BE, EP), lambda i: (i, 0)))
    in_specs = [pl.BlockSpec((_BE, UW), lambda i: (i, 0)),
                pl.BlockSpec((_BE, UW), lambda i: (i, 0)),
                e_spec] + [_full_spec(a.shape) for a in w]
    return pl.pallas_call(
        body,
        grid=(nblk,),
        in_specs=in_specs,
        out_specs=[pl.BlockSpec((_BE, HIDDEN), lambda i: (i, 0)),
                   pl.BlockSpec((_BE, EP), lambda i: (i, 0))],
        out_shape=[jax.ShapeDtypeStruct((NE_PAD, HIDDEN), jnp.float32),
                   jax.ShapeDtypeStruct((NE_PAD, EP), jnp.float32)],
    )(gs, gd, e, *w)


_BN = 1000


def _update_stage(u, p, c0, c1, w):
    nblk = N_NODES // _BN
    in_specs = [pl.BlockSpec((_BN, UW), lambda i: (i, 0)),
                pl.BlockSpec((_BN, HIDDEN), lambda i: (i, 0)),
                pl.BlockSpec((_BN, HIDDEN), lambda i: (i, 0)),
                pl.BlockSpec((_BN, HIDDEN), lambda i: (i, 0)),
                pl.BlockSpec((_BN, HIDDEN), lambda i: (i, 0))] + \
               [_full_spec(a.shape) for a in w]
    return pl.pallas_call(
        _update_body,
        grid=(nblk,),
        in_specs=in_specs,
        out_specs=pl.BlockSpec((_BN, UW), lambda i: (i, 0)),
        out_shape=jax.ShapeDtypeStruct((N_NODES, UW), jnp.float32),
    )(u, p[0], p[1], c0, c1, *w)


def _final_stage(u, w):
    specs = [_full_spec(a.shape) for a in (u,) + w]
    return pl.pallas_call(
        _final_body,
        grid=(1,),
        in_specs=specs,
        out_specs=_full_spec((1, LATENT)),
        out_shape=jax.ShapeDtypeStruct((1, LATENT), jnp.float32),
    )(u, *w)


def _sc_mesh():
    return plsc.VectorSubcoreMesh(core_axis_name="c", subcore_axis_name="s",
                                  num_cores=2, num_subcores=16)


_GC = 256
_IRT = PER_TILE // 128
_NCHK = PER_TILE // _GC


@functools.cache
def _sc_gather_kernel():
    @functools.partial(
        pl.kernel,
        out_type=(jax.ShapeDtypeStruct((NE_PAD, UW), jnp.float32),
                  jax.ShapeDtypeStruct((NE_PAD, UW), jnp.float32)),
        mesh=_sc_mesh(),
        scratch_types=[pltpu.VMEM((2 * _IRT, 128), jnp.int32),
                       pltpu.VMEM((_GC, UW), jnp.float32),
                       pltpu.VMEM((_GC, UW), jnp.float32),
                       pltpu.SemaphoreType.DMA,
                       pltpu.SemaphoreType.DMA],
    )
    def k(u_hbm, src_hbm, dst_hbm, gs_hbm, gd_hbm, ibuf, buf_a, buf_b, sg, sw):
        c = lax.axis_index("c")
        s = lax.axis_index("s")
        w = c * 16 + s
        base_e = pl.multiple_of(w * PER_TILE, PER_TILE)
        irow = pl.multiple_of(w * _IRT, _IRT)
        pltpu.sync_copy(src_hbm.at[pl.ds(irow, _IRT)], ibuf.at[pl.ds(0, _IRT)])
        pltpu.sync_copy(dst_hbm.at[pl.ds(irow, _IRT)], ibuf.at[pl.ds(_IRT, _IRT)])

        bufs = (buf_a, buf_b)
        outs = (gs_hbm, gd_hbm)

        def fire(kk):
            phase, j = divmod(kk, _NCHK)
            ib = phase * _IRT + j * (_GC // 128)
            buf = bufs[kk % 2]
            return [
                pltpu.async_copy(u_hbm.at[ibuf.at[ib + t]],
                                 buf.at[pl.ds(t * 128, 128)], sg)
                for t in range(_GC // 128)
            ]

        pending_g = fire(0)
        pending_w = None
        for kk in range(2 * _NCHK):
            phase, j = divmod(kk, _NCHK)
            buf = bufs[kk % 2]
            if pending_w is not None:
                pending_w.wait()
            next_g = fire(kk + 1) if kk + 1 < 2 * _NCHK else None
            for h in pending_g:
                h.wait()
            rows = pl.multiple_of(base_e + j * _GC, _GC)
            pending_w = pltpu.async_copy(buf, outs[phase].at[pl.ds(rows, _GC)], sw)
            pending_g = next_g
        pending_w.wait()

    return k


def _sc_gather(u, src2, dst2):
    return _sc_gather_kernel()(u, src2, dst2)


def _seg_sum(vals, idx):
    half = NE_PAD // 2
    p0 = jax.ops.segment_sum(vals[:half], idx[:half], num_segments=NPAD)
    p1 = jax.ops.segment_sum(vals[half:], idx[half:], num_segments=NPAD)
    return p0, p1


def _row(v, width=None):
    v = v.reshape(1, -1)
    if width is not None and v.shape[1] < width:
        v = jnp.pad(v, ((0, 0), (0, width - v.shape[1])))
    return v


def _pad_rows(w, rows):
    return jnp.pad(w, ((0, rows - w.shape[0]), (0, 0)))


def _pad_cols(w, cols):
    return jnp.pad(w, ((0, 0), (0, cols - w.shape[1])))


def kernel(x, edge_index, params):
    p = params
    npad = NE_PAD - N_EDGES
    src = jnp.concatenate([edge_index[0], jnp.zeros((npad,), jnp.int32)])
    dst = jnp.concatenate([edge_index[1],
                           jnp.full((npad,), N_NODES, jnp.int32)])
    src2 = src.reshape(NE_PAD // 128, 128)
    dst2 = dst.reshape(NE_PAD // 128, 128)

    ones_h = jnp.ones((NE_PAD, HIDDEN), jnp.float32)

    enc_w = (p['mlp0_w1'], _row(p['mlp0_b1']),
             _pad_cols(p['mlp0_w2'], UW), _row(p['mlp0_b2'], UW),
             _row(p['lnm0_g'], UW), _row(p['lnm0_b'], UW),
             _row(p['mlp1_b1']), _pad_cols(p['mlp1_w2'], EP),
             _row(p['mlp1_b2'], EP), _row(p['lnm1_g'], EP), _row(p['lnm1_b'], EP))

    u, e0 = _node_enc(x, enc_w)

    _idx = dst2.reshape(-1)
    c0, c1 = _seg_sum(ones_h, _idx)

    e = e0
    for li, lyr in enumerate(p['layers']):
        gs, gd = _sc_gather(u, src2, dst2)
        ew = (_pad_rows(lyr['msg_w1'][:HIDDEN], UW),
              _pad_rows(lyr['msg_w1'][HIDDEN:2 * HIDDEN], UW),
              _pad_rows(lyr['msg_w1'][2 * HIDDEN:], EP),
              _row(lyr['msg_b1']), lyr['msg_w2'], _row(lyr['msg_b2']),
              _pad_rows(lyr['edg_w1'][:HIDDEN], UW),
              _pad_rows(lyr['edg_w1'][HIDDEN:2 * HIDDEN], UW),
              _pad_rows(lyr['edg_w1'][2 * HIDDEN:], EP),
              _row(lyr['edg_b1']), _pad_cols(lyr['edg_w2'], EP),
              _row(lyr['edg_b2'], EP),
              _row(lyr['ln_e_g'], EP), _row(lyr['ln_e_b'], EP))
        m, e = _edge_stage(gs, gd, e, ew, first=(li == 0))
        p0, p1 = _seg_sum(m, _idx)
        uw = (_pad_rows(lyr['upd_w1'][:HIDDEN], UW), lyr['upd_w1'][HIDDEN:],
              _row(lyr['upd_b1']), _pad_cols(lyr['upd_w2'], UW),
              _row(lyr['upd_b2'], UW),
              _row(lyr['ln_n_g'], UW), _row(lyr['ln_n_b'], UW))
        u = _update_stage(u, (p0, p1), c0, c1, uw)

    fw = (_pad_rows(p['mlp2_w1'], UW), _row(p['mlp2_b1']),
          p['mlp2_w2'], _row(p['mlp2_b2']),
          _row(p['lnm2_g']), _row(p['lnm2_b']),
          p['out_w1'], _row(p['out_b1']), p['out_w2'], _row(p['out_b2']))
    return _final_stage(u, fw)


# speedup vs baseline: 1.0301x; 1.0047x over previous
import functools

import jax
import jax.numpy as jnp
from jax import lax
from jax.experimental import pallas as pl
from jax.experimental.pallas import tpu as pltpu
from jax.experimental.pallas import tpu_sc as plsc

N_NODES = 10000
N_EDGES = 320000
HIDDEN = 64
UW = 128
EDGE_DIM = 2
EP = 8
LATENT = 32

NW = 32
NE_PAD = 327680
NPAD = 10240
PER_TILE = NE_PAD // NW
CH = 512
NCH = CH // 128
N_OUTER = PER_TILE // CH
CNT_W = 16

_EPS = 1e-5
_PREC = lax.Precision.HIGHEST


def _relu(v):
    return jnp.maximum(v, 0.0)


def _dot(a, b):
    return jnp.dot(a, b, precision=_PREC, preferred_element_type=jnp.float32)


def _ln_rows(t, g, b, width):
    s = jnp.sum(t, axis=-1, keepdims=True)
    mean = s / width
    var = jnp.sum(t * t, axis=-1, keepdims=True) / width - mean * mean
    return (t - mean) * lax.rsqrt(var + _EPS) * g + b


def _pack_hi_lo(x):
    return jnp.concatenate([x, jnp.zeros_like(x)], axis=-1)


def _unpack_hi_lo(p):
    return p[:, :HIDDEN]


def _node_enc_body(x_ref, w1_ref, b1_ref, w2_ref, b2_ref, g_ref, bb_ref,
                   m1b1_ref, m1w2_ref, m1b2_ref, eg_ref, ebb_ref,
                   u_ref, up_ref, e0_ref):
    x = x_ref[...]
    h = _relu(_dot(x, w1_ref[...]) + b1_ref[...])
    t = _dot(h, w2_ref[...]) + b2_ref[...]
    u = _ln_rows(t, g_ref[...], bb_ref[...], HIDDEN)
    u_ref[...] = u
    up_ref[...] = _pack_hi_lo(u)
    h0 = _relu(m1b1_ref[...])
    e0 = _dot(h0, m1w2_ref[...]) + m1b2_ref[...]
    e0_ref[...] = _ln_rows(e0, eg_ref[...], ebb_ref[...], EDGE_DIM)


def _edge_body_common(gs, gd, e,
                      wa_ref, wb_ref, wc_ref, b1_ref, w2_ref, b2_ref,
                      ea_ref, eb_ref, ec_ref, f1_ref, ew2_ref, f2_ref,
                      g_ref, bb_ref, m_ref, eo_ref):
    pre = _dot(gs, wa_ref[...]) + _dot(gd, wb_ref[...]) + _dot(e, wc_ref[...]) + b1_ref[...]
    m_ref[...] = _dot(_relu(pre), w2_ref[...]) + b2_ref[...]
    pre2 = _dot(gs, ea_ref[...]) + _dot(gd, eb_ref[...]) + _dot(e, ec_ref[...]) + f1_ref[...]
    q = _dot(_relu(pre2), ew2_ref[...]) + f2_ref[...]
    eo_ref[...] = _ln_rows(e + q, g_ref[...], bb_ref[...], EDGE_DIM)


def _edge_body(gs_ref, gd_ref, e_ref, *rest):
    _edge_body_common(_unpack_hi_lo(gs_ref[...]), _unpack_hi_lo(gd_ref[...]),
                      e_ref[...], *rest)


def _edge0_body(gs_ref, gd_ref, e0_ref, *rest):
    e = jnp.broadcast_to(e0_ref[...], (gs_ref.shape[0], EP))
    _edge_body_common(_unpack_hi_lo(gs_ref[...]), _unpack_hi_lo(gd_ref[...]),
                      e, *rest)


def _update_body(u_ref, p0_ref, p1_ref, c0_ref, c1_ref,
                 ua_ref, ub_ref, b1_ref, w2_ref, b2_ref, g_ref, bb_ref,
                 out_ref, outp_ref):
    u = u_ref[...]
    agg = p0_ref[...] + p1_ref[...]
    cnt = jnp.maximum(c0_ref[...][:, 0:1] + c1_ref[...][:, 0:1], 1.0)
    mean_agg = agg / cnt
    h = _relu(_dot(u, ua_ref[...]) + _dot(mean_agg, ub_ref[...]) + b1_ref[...])
    t = u + _dot(h, w2_ref[...]) + b2_ref[...]
    un = _ln_rows(t, g_ref[...], bb_ref[...], HIDDEN)
    out_ref[...] = un
    outp_ref[...] = _pack_hi_lo(un)


def _final_body(u_ref, w1_ref, b1_ref, w2_ref, b2_ref, g_ref, bb_ref,
                ow1_ref, ob1_ref, ow2_ref, ob2_ref, z_ref):
    u = u_ref[...]
    h = _relu(_dot(u, w1_ref[...]) + b1_ref[...])
    u3 = _ln_rows(_dot(h, w2_ref[...]) + b2_ref[...], g_ref[...], bb_ref[...], HIDDEN)
    z = jnp.mean(u3, axis=0, keepdims=True)
    h2 = _relu(_dot(z, ow1_ref[...]) + ob1_ref[...])
    z_ref[...] = _dot(h2, ow2_ref[...]) + ob2_ref[...]


def _full_spec(shape):
    return pl.BlockSpec(shape, lambda *_: tuple(0 for _ in shape))


def _node_enc(x, w):
    specs = [_full_spec(a.shape) for a in (x,) + w]
    return pl.pallas_call(
        _node_enc_body,
        grid=(1,),
        in_specs=specs,
        out_specs=[_full_spec((N_NODES, HIDDEN)), _full_spec((N_NODES, UW)),
                   _full_spec((1, EP))],
        out_shape=[jax.ShapeDtypeStruct((N_NODES, HIDDEN), jnp.float32),
                   jax.ShapeDtypeStruct((N_NODES, UW), jnp.float32),
                   jax.ShapeDtypeStruct((1, EP), jnp.float32)],
    )(x, *w)


_BE = 2048


def _edge_stage(gs, gd, e, w, first):
    body = _edge0_body if first else _edge_body
    nblk = NE_PAD // _BE
    e_spec = (_full_spec((1, EP)) if first
              else pl.BlockSpec((_BE, EP), lambda i: (i, 0)))
    in_specs = [pl.BlockSpec((_BE, UW), lambda i: (i, 0)),
                pl.BlockSpec((_BE, UW), lambda i: (i, 0)),
                e_spec] + [_full_spec(a.shape) for a in w]
    return pl.pallas_call(
        body,
        grid=(nblk,),
        in_specs=in_specs,
        out_specs=[pl.BlockSpec((_BE, HIDDEN), lambda i: (i, 0)),
                   pl.BlockSpec((_BE, EP), lambda i: (i, 0))],
        out_shape=[jax.ShapeDtypeStruct((NE_PAD, HIDDEN), jnp.float32),
                   jax.ShapeDtypeStruct((NE_PAD, EP), jnp.float32)],
    )(gs, gd, e, *w)


_BN = 1000


def _update_stage(u, p, c0, c1, w):
    nblk = N_NODES // _BN
    in_specs = [pl.BlockSpec((_BN, HIDDEN), lambda i: (i, 0)),
                pl.BlockSpec((_BN, HIDDEN), lambda i: (i, 0)),
                pl.BlockSpec((_BN, HIDDEN), lambda i: (i, 0)),
                pl.BlockSpec((_BN, HIDDEN), lambda i: (i, 0)),
                pl.BlockSpec((_BN, HIDDEN), lambda i: (i, 0))] + \
               [_full_spec(a.shape) for a in w]
    return pl.pallas_call(
        _update_body,
        grid=(nblk,),
        in_specs=in_specs,
        out_specs=[pl.BlockSpec((_BN, HIDDEN), lambda i: (i, 0)),
                   pl.BlockSpec((_BN, UW), lambda i: (i, 0))],
        out_shape=[jax.ShapeDtypeStruct((N_NODES, HIDDEN), jnp.float32),
                   jax.ShapeDtypeStruct((N_NODES, UW), jnp.float32)],
    )(u, p[0], p[1], c0, c1, *w)


def _final_stage(u, w):
    specs = [_full_spec(a.shape) for a in (u,) + w]
    return pl.pallas_call(
        _final_body,
        grid=(1,),
        in_specs=specs,
        out_specs=_full_spec((1, LATENT)),
        out_shape=jax.ShapeDtypeStruct((1, LATENT), jnp.float32),
    )(u, *w)


def _sc_mesh():
    return plsc.VectorSubcoreMesh(core_axis_name="c", subcore_axis_name="s",
                                  num_cores=2, num_subcores=16)


_GC = 256
_IRT = PER_TILE // 128
_NCHK = PER_TILE // _GC


@functools.cache
def _sc_gather_kernel():
    @functools.partial(
        pl.kernel,
        out_type=(jax.ShapeDtypeStruct((NE_PAD, UW), jnp.float32),
                  jax.ShapeDtypeStruct((NE_PAD, UW), jnp.float32)),
        mesh=_sc_mesh(),
        scratch_types=[pltpu.VMEM((2 * _IRT, 128), jnp.int32),
                       pltpu.VMEM((_GC, UW), jnp.float32),
                       pltpu.VMEM((_GC, UW), jnp.float32),
                       pltpu.SemaphoreType.DMA,
                       pltpu.SemaphoreType.DMA],
    )
    def k(u_hbm, src_hbm, dst_hbm, gs_hbm, gd_hbm, ibuf, buf_a, buf_b, sg, sw):
        c = lax.axis_index("c")
        s = lax.axis_index("s")
        w = c * 16 + s
        base_e = pl.multiple_of(w * PER_TILE, PER_TILE)
        irow = pl.multiple_of(w * _IRT, _IRT)
        pltpu.sync_copy(src_hbm.at[pl.ds(irow, _IRT)], ibuf.at[pl.ds(0, _IRT)])
        pltpu.sync_copy(dst_hbm.at[pl.ds(irow, _IRT)], ibuf.at[pl.ds(_IRT, _IRT)])

        bufs = (buf_a, buf_b)
        outs = (gs_hbm, gd_hbm)

        def fire(kk):
            phase, j = divmod(kk, _NCHK)
            ib = phase * _IRT + j * (_GC // 128)
            buf = bufs[kk % 2]
            return [
                pltpu.async_copy(u_hbm.at[ibuf.at[ib + t]],
                                 buf.at[pl.ds(t * 128, 128)], sg)
                for t in range(_GC // 128)
            ]

        pending_g = fire(0)
        pending_w = None
        for kk in range(2 * _NCHK):
            phase, j = divmod(kk, _NCHK)
            buf = bufs[kk % 2]
            if pending_w is not None:
                pending_w.wait()
            next_g = fire(kk + 1) if kk + 1 < 2 * _NCHK else None
            for h in pending_g:
                h.wait()
            rows = pl.multiple_of(base_e + j * _GC, _GC)
            pending_w = pltpu.async_copy(buf, outs[phase].at[pl.ds(rows, _GC)], sw)
            pending_g = next_g
        pending_w.wait()

    return k


def _sc_gather(u, src2, dst2):
    return _sc_gather_kernel()(u, src2, dst2)


def _seg_sum(vals, idx):
    half = NE_PAD // 2
    p0 = jax.ops.segment_sum(vals[:half], idx[:half], num_segments=NPAD)
    p1 = jax.ops.segment_sum(vals[half:], idx[half:], num_segments=NPAD)
    return p0, p1


def _row(v, width=None):
    v = v.reshape(1, -1)
    if width is not None and v.shape[1] < width:
        v = jnp.pad(v, ((0, 0), (0, width - v.shape[1])))
    return v


def _pad_rows(w, rows):
    return jnp.pad(w, ((0, rows - w.shape[0]), (0, 0)))


def _pad_cols(w, cols):
    return jnp.pad(w, ((0, 0), (0, cols - w.shape[1])))


def kernel(x, edge_index, params):
    p = params
    npad = NE_PAD - N_EDGES
    src = jnp.concatenate([edge_index[0], jnp.zeros((npad,), jnp.int32)])
    dst = jnp.concatenate([edge_index[1],
                           jnp.full((npad,), N_NODES, jnp.int32)])
    src2 = src.reshape(NE_PAD // 128, 128)
    dst2 = dst.reshape(NE_PAD // 128, 128)

    ones_h = jnp.ones((NE_PAD, HIDDEN), jnp.float32)

    enc_w = (p['mlp0_w1'], _row(p['mlp0_b1']),
             p['mlp0_w2'], _row(p['mlp0_b2']),
             _row(p['lnm0_g']), _row(p['lnm0_b']),
             _row(p['mlp1_b1']), _pad_cols(p['mlp1_w2'], EP),
             _row(p['mlp1_b2'], EP), _row(p['lnm1_g'], EP), _row(p['lnm1_b'], EP))

    u, upk, e0 = _node_enc(x, enc_w)

    _idx = dst2.reshape(-1)
    c0, c1 = _seg_sum(ones_h, _idx)

    e = e0
    for li, lyr in enumerate(p['layers']):
        gs, gd = _sc_gather(upk, src2, dst2)
        ew = (lyr['msg_w1'][:HIDDEN], lyr['msg_w1'][HIDDEN:2 * HIDDEN],
              _pad_rows(lyr['msg_w1'][2 * HIDDEN:], EP),
              _row(lyr['msg_b1']), lyr['msg_w2'], _row(lyr['msg_b2']),
              lyr['edg_w1'][:HIDDEN], lyr['edg_w1'][HIDDEN:2 * HIDDEN],
              _pad_rows(lyr['edg_w1'][2 * HIDDEN:], EP),
              _row(lyr['edg_b1']), _pad_cols(lyr['edg_w2'], EP),
              _row(lyr['edg_b2'], EP),
              _row(lyr['ln_e_g'], EP), _row(lyr['ln_e_b'], EP))
        m, e = _edge_stage(gs, gd, e, ew, first=(li == 0))
        p0, p1 = _seg_sum(m, _idx)
        uw = (lyr['upd_w1'][:HIDDEN], lyr['upd_w1'][HIDDEN:],
              _row(lyr['upd_b1']), lyr['upd_w2'], _row(lyr['upd_b2']),
              _row(lyr['ln_n_g']), _row(lyr['ln_n_b']))
        u, upk = _update_stage(u, (p0, p1), c0, c1, uw)

    fw = (p['mlp2_w1'], _row(p['mlp2_b1']),
          p['mlp2_w2'], _row(p['mlp2_b2']),
          _row(p['lnm2_g']), _row(p['lnm2_b']),
          p['out_w1'], _row(p['out_b1']), p['out_w2'], _row(p['out_b2']))
    return _final_stage(u, fw)
